# baseline (speedup 1.0000x reference)
import functools

import jax
import jax.numpy as jnp
import numpy as np
from jax import lax
from jax.experimental import pallas as pl
from jax.experimental.pallas import tpu as pltpu
from jax.experimental.pallas import tpu_sc as plsc

N = 10000
E = 160000
FEAT = 128
HID = 128
RATIO = 0.5

NW = 32
EP = 163840
EPW = EP // NW
DEGB = 10240

def _sc_mesh():
    return plsc.VectorSubcoreMesh(core_axis_name="c", subcore_axis_name="s")


def _sc_wid():
    return lax.axis_index("c") * 16 + lax.axis_index("s")


def _deg_sc_kernel(dst_hbm, out_hbm, dstv, onesv, zbuf, degsh):
    c = lax.axis_index("c")
    s = lax.axis_index("s")
    wid = c * 16 + s

    def zfill(i, _):
        zbuf[pl.ds(i * 16, 16)] = jnp.zeros((16,), jnp.float32)
        return 0
    lax.fori_loop(0, zbuf.shape[0] // 16, zfill, 0)

    def ofill(i, _):
        onesv[pl.ds(i * 16, 16)] = jnp.ones((16,), jnp.float32)
        return 0
    lax.fori_loop(0, EPW // 16, ofill, 0)

    sl = DEGB // 16
    pltpu.sync_copy(zbuf, degsh.at[pl.ds(s * sl, sl)])
    pltpu.sync_copy(dst_hbm.at[wid], dstv)
    plsc.subcore_barrier()
    pltpu.sync_copy(onesv, degsh.at[dstv], add=True)
    plsc.subcore_barrier()
    pltpu.sync_copy(degsh.at[pl.ds(s * sl, sl)], out_hbm.at[c, pl.ds(s * sl, sl)])


def _deg_partials(dst_pad):
    k = pl.kernel(
        _deg_sc_kernel,
        out_type=jax.ShapeDtypeStruct((2, DEGB), jnp.float32),
        mesh=_sc_mesh(),
        scratch_types=[
            pltpu.VMEM((EPW,), jnp.int32),
            pltpu.VMEM((EPW,), jnp.float32),
            pltpu.VMEM((DEGB // 16,), jnp.float32),
            pltpu.VMEM_SHARED((DEGB,), jnp.float32),
        ],
    )
    return k(dst_pad.reshape(NW, EPW))


def _eprod_sc_kernel(src_hbm, dst_hbm, tab_hbm, out_hbm, sv, dv, av, bv, ov,
                     tabsh):
    wid = _sc_wid()
    s = lax.axis_index("s")
    sl = DEGB // 16
    pltpu.sync_copy(tab_hbm.at[pl.ds(s * sl, sl)], tabsh.at[pl.ds(s * sl, sl)])
    pltpu.sync_copy(src_hbm.at[wid], sv)
    pltpu.sync_copy(dst_hbm.at[wid], dv)
    plsc.subcore_barrier()
    pltpu.sync_copy(tabsh.at[sv], av)
    pltpu.sync_copy(tabsh.at[dv], bv)

    def body(i, _):
        sl = pl.ds(i * 16, 16)
        ov[sl] = av[sl] * bv[sl]
        return 0
    lax.fori_loop(0, EPW // 16, body, 0)
    pltpu.sync_copy(ov, out_hbm.at[wid])


def _edge_products(src_pad, dst_pad, table):
    k = pl.kernel(
        _eprod_sc_kernel,
        out_type=jax.ShapeDtypeStruct((NW, EPW), jnp.float32),
        mesh=_sc_mesh(),
        scratch_types=[
            pltpu.VMEM((EPW,), jnp.int32),
            pltpu.VMEM((EPW,), jnp.int32),
            pltpu.VMEM((EPW,), jnp.float32),
            pltpu.VMEM((EPW,), jnp.float32),
            pltpu.VMEM((EPW,), jnp.float32),
            pltpu.VMEM_SHARED((DEGB,), jnp.float32),
        ],
    )
    return k(src_pad.reshape(NW, EPW), dst_pad.reshape(NW, EPW), table)


def _permvals_sc_kernel(rank_hbm, s_hbm, nid_hbm, perm_hbm, vals_hbm,
                        rv, sv, nv, iv):
    wid = _sc_wid()
    kslots = perm_hbm.shape[0] - 120
    pltpu.sync_copy(rank_hbm.at[wid], rv)
    pltpu.sync_copy(s_hbm.at[wid], sv)
    pltpu.sync_copy(nid_hbm.at[wid], nv)

    def body(i, _):
        sl = pl.ds(i * 16, 16)
        r = rv[sl]
        nid = nv[sl]
        idx = jnp.where(r < kslots, r, kslots + lax.rem(nid, 120))
        rv[sl] = idx
        return 0
    lax.fori_loop(0, rv.shape[0] // 16, body, 0)
    pltpu.sync_copy(nv, perm_hbm.at[rv])
    pltpu.sync_copy(sv, vals_hbm.at[rv])


def _perm_and_vals(rank, s, k):
    npd = rank.shape[0] if rank.shape[0] % NW == 0 else DEGB
    chunk = DEGB // NW
    rank_p = jnp.concatenate(
        [rank, jnp.full((DEGB - rank.shape[0],), 2 * DEGB, jnp.int32)])
    s_p = jnp.concatenate(
        [s, jnp.zeros((DEGB - s.shape[0],), jnp.float32)])
    nid = jnp.arange(DEGB, dtype=jnp.int32)
    kp = pl.kernel(
        _permvals_sc_kernel,
        out_type=(jax.ShapeDtypeStruct((k + 120,), jnp.int32),
                  jax.ShapeDtypeStruct((k + 120,), jnp.float32)),
        mesh=_sc_mesh(),
        scratch_types=[
            pltpu.VMEM((chunk,), jnp.int32),
            pltpu.VMEM((chunk,), jnp.float32),
            pltpu.VMEM((chunk,), jnp.int32),
            pltpu.VMEM((chunk,), jnp.int32),
        ],
    )
    perm_p, vals_p = kp(rank_p.reshape(NW, chunk), s_p.reshape(NW, chunk),
                        nid.reshape(NW, chunk))
    return perm_p[:k], vals_p[:k]


def _relabel_sc_kernel(src_hbm, dst_hbm, keptm_hbm, m0z_hbm,
                       w1_hbm, src1_hbm, dst1_hbm, degout_hbm,
                       sv, dv, ks, kd, ms, md, wv, zbuf, degsh, keptsh, mzsh):
    c = lax.axis_index("c")
    s = lax.axis_index("s")
    wid = c * 16 + s

    def zfill(i, _):
        zbuf[pl.ds(i * 16, 16)] = jnp.zeros((16,), jnp.float32)
        return 0
    lax.fori_loop(0, zbuf.shape[0] // 16, zfill, 0)
    sl = DEGB // 16
    pltpu.sync_copy(zbuf, degsh.at[pl.ds(s * sl, sl)])
    pltpu.sync_copy(keptm_hbm.at[pl.ds(s * sl, sl)],
                    keptsh.at[pl.ds(s * sl, sl)])
    pltpu.sync_copy(m0z_hbm.at[pl.ds(s * sl, sl)], mzsh.at[pl.ds(s * sl, sl)])

    pltpu.sync_copy(src_hbm.at[wid], sv)
    pltpu.sync_copy(dst_hbm.at[wid], dv)
    plsc.subcore_barrier()
    pltpu.sync_copy(keptsh.at[sv], ks)
    pltpu.sync_copy(keptsh.at[dv], kd)
    pltpu.sync_copy(mzsh.at[sv], ms)
    pltpu.sync_copy(mzsh.at[dv], md)

    def body(i, _):
        q = pl.ds(i * 16, 16)
        wv[q] = ks[q] * kd[q]
        return 0
    lax.fori_loop(0, EPW // 16, body, 0)
    plsc.subcore_barrier()
    pltpu.sync_copy(wv, degsh.at[md], add=True)
    pltpu.sync_copy(wv, w1_hbm.at[wid])
    pltpu.sync_copy(ms, src1_hbm.at[wid])
    pltpu.sync_copy(md, dst1_hbm.at[wid])
    plsc.subcore_barrier()
    pltpu.sync_copy(degsh.at[pl.ds(s * sl, sl)], degout_hbm.at[c, pl.ds(s * sl, sl)])


def _relabel_edges(src_pad, dst_pad, keptm_ext, m0z_ext):
    k = pl.kernel(
        _relabel_sc_kernel,
        out_type=(jax.ShapeDtypeStruct((NW, EPW), jnp.float32),
                  jax.ShapeDtypeStruct((NW, EPW), jnp.int32),
                  jax.ShapeDtypeStruct((NW, EPW), jnp.int32),
                  jax.ShapeDtypeStruct((2, DEGB), jnp.float32)),
        mesh=_sc_mesh(),
        scratch_types=[
            pltpu.VMEM((EPW,), jnp.int32),
            pltpu.VMEM((EPW,), jnp.int32),
            pltpu.VMEM((EPW,), jnp.float32),
            pltpu.VMEM((EPW,), jnp.float32),
            pltpu.VMEM((EPW,), jnp.int32),
            pltpu.VMEM((EPW,), jnp.int32),
            pltpu.VMEM((EPW,), jnp.float32),
            pltpu.VMEM((DEGB // 16,), jnp.float32),
            pltpu.VMEM_SHARED((DEGB,), jnp.float32),
            pltpu.VMEM_SHARED((DEGB,), jnp.float32),
            pltpu.VMEM_SHARED((DEGB,), jnp.int32),
        ],
    )
    return k(src_pad.reshape(NW, EPW), dst_pad.reshape(NW, EPW),
             keptm_ext, m0z_ext)


def _gather_rows_sc_kernel(tab_hbm, idx_hbm, out_hbm, iv, rows, sem):
    wid = _sc_wid()
    chunk = iv.shape[0]
    pltpu.sync_copy(idx_hbm.at[wid], iv)
    pltpu.async_copy(tab_hbm.at[iv], rows, sem).wait()
    pltpu.sync_copy(rows, out_hbm.at[pl.ds(wid * chunk, chunk)])


def _gather_rows(tab, idx_ext):
    chunk = idx_ext.shape[0] // NW
    k = pl.kernel(
        _gather_rows_sc_kernel,
        out_type=jax.ShapeDtypeStruct((idx_ext.shape[0], HID), jnp.float32),
        mesh=_sc_mesh(),
        scratch_types=[
            pltpu.VMEM((chunk,), jnp.int32),
            pltpu.VMEM((chunk, HID), jnp.float32),
            pltpu.SemaphoreType.DMA,
        ],
    )
    return k(tab, idx_ext.reshape(NW, chunk))


def _rowagg_sc_kernel(h_hbm, src_hbm, dst_hbm, out_hbm,
                      srcv, dstv, rows, zbuf, accsh):
    c = lax.axis_index("c")
    s = lax.axis_index("s")
    wid = c * 16 + s

    def zfill(i, _):
        zbuf[i // 8, pl.ds((i % 8) * 16, 16)] = jnp.zeros((16,), jnp.float32)
        return 0
    lax.fori_loop(0, 16 * 8, zfill, 0)

    def zcopy(i, _):
        pltpu.sync_copy(zbuf, accsh.at[pl.ds(s * 640 + i * 16, 16)])
        return 0
    lax.fori_loop(0, 40, zcopy, 0)

    pltpu.sync_copy(src_hbm.at[wid], srcv)
    pltpu.sync_copy(dst_hbm.at[wid], dstv)
    plsc.subcore_barrier()

    def chunk(i, _):
        pltpu.sync_copy(h_hbm.at[srcv.at[i]], rows)
        pltpu.sync_copy(rows, accsh.at[dstv.at[i]], add=True)
        return 0
    lax.fori_loop(0, srcv.shape[0], chunk, 0)
    plsc.subcore_barrier()
    pltpu.sync_copy(accsh.at[pl.ds(s * 640, 640)],
                    out_hbm.at[c, pl.ds(s * 640, 640)])


def _row_scatter_sum(h_ext, src_pad, dst_pad):
    nchunk, csz = 40, 128
    k = pl.kernel(
        _rowagg_sc_kernel,
        out_type=jax.ShapeDtypeStruct((2, DEGB, HID), jnp.float32),
        mesh=_sc_mesh(),
        scratch_types=[
            pltpu.VMEM((nchunk, csz), jnp.int32),
            pltpu.VMEM((nchunk, csz), jnp.int32),
            pltpu.VMEM((csz, HID), jnp.float32),
            pltpu.VMEM((16, HID), jnp.float32),
            pltpu.VMEM_SHARED((DEGB, HID), jnp.float32),
        ],
    )
    return k(h_ext, src_pad.reshape(NW, nchunk, csz),
             dst_pad.reshape(NW, nchunk, csz))


def _merge_kernel(e1_ref, e2u_ref, wl_ref, invs_ref, alpha_ref,
                  h2p_ref, lp_ref):
    a0 = alpha_ref[0]
    a1 = alpha_ref[1]
    e1b = e1_ref[...]
    emb = a0 * e1b + a1 * e2u_ref[...]
    lp_ref[...] = jnp.sum(1.0 + e1b - emb * emb - jnp.exp(e1b), axis=0,
                          keepdims=True).reshape(1, 1, HID)
    h2 = jnp.dot(emb, wl_ref[...], preferred_element_type=jnp.float32)
    h2p_ref[...] = h2 * invs_ref[0, 0, :].reshape(e1b.shape[0], 1)


def _merge_scale(e1, e2_up, Wl, invs, alpha, blk=400):
    grid = N // blk
    h2p, lp = pl.pallas_call(
        _merge_kernel,
        grid=(grid,),
        in_specs=[pl.BlockSpec((blk, HID), lambda i: (i, 0)),
                  pl.BlockSpec((blk, HID), lambda i: (i, 0)),
                  pl.BlockSpec((HID, HID), lambda i: (0, 0)),
                  pl.BlockSpec((1, 1, blk), lambda i: (i, 0, 0)),
                  pl.BlockSpec(memory_space=pltpu.SMEM)],
        out_specs=[pl.BlockSpec((blk, HID), lambda i: (i, 0)),
                   pl.BlockSpec((1, 1, HID), lambda i: (i, 0, 0))],
        out_shape=[jax.ShapeDtypeStruct((N, HID), jnp.float32),
                   jax.ShapeDtypeStruct((grid, 1, HID), jnp.float32)],
    )(e1, e2_up, Wl, invs.reshape(grid, 1, blk), alpha)
    return h2p, lp


def _finalize_kernel(pa_ref, pb_ref, invs_ref, bl_ref, o_ref):
    blk = o_ref.shape[0]
    v = ((pa_ref[...] + pb_ref[...]) * invs_ref[0, 0, :].reshape(blk, 1)
         + bl_ref[...])
    n = jnp.sqrt(jnp.sum(v * v, axis=1, keepdims=True))
    o_ref[...] = v / jnp.maximum(n, 1e-12)


def _finalize(pa, pb, invs, bl, blk=400):
    grid = N // blk
    return pl.pallas_call(
        _finalize_kernel,
        grid=(grid,),
        in_specs=[pl.BlockSpec((blk, HID), lambda i: (i, 0)),
                  pl.BlockSpec((blk, HID), lambda i: (i, 0)),
                  pl.BlockSpec((1, 1, blk), lambda i: (i, 0, 0)),
                  pl.BlockSpec((1, HID), lambda i: (0, 0))],
        out_specs=pl.BlockSpec((blk, HID), lambda i: (i, 0)),
        out_shape=jax.ShapeDtypeStruct((N, HID), jnp.float32),
    )(pa, pb, invs.reshape(grid, 1, blk), bl.reshape(1, HID))


def _normalize(x, axis=-1, eps=1e-12):
    n = jnp.linalg.norm(x, axis=axis, keepdims=True)
    return x / jnp.maximum(n, eps)


def _gcn(x, W, b, src, dst, w, n):
    h = x @ W
    deg = jax.ops.segment_sum(w, dst, num_segments=n)
    deg = jnp.clip(deg, 1.0, None)
    norm = w / jnp.sqrt(deg[src] * deg[dst])
    agg = jax.ops.segment_sum(h[src] * norm[:, None], dst, num_segments=n)
    return agg + b


def _pool_meta(emb, p, n, ratio):
    score = jax.nn.sigmoid(emb @ p)
    k = int(np.ceil(ratio * n))
    vals, perm = jax.lax.top_k(score, k)
    kept = jnp.zeros((n,), jnp.float32).at[perm].set(1.0)
    mapping = jnp.zeros((n,), jnp.int32).at[perm].set(jnp.arange(k, dtype=jnp.int32))
    return vals, perm, kept, mapping, k


def _rank_kernel(sfull_ref, srow_ref, o_ref):
    i = pl.program_id(0)
    s_r = srow_ref[0, 0, :].reshape(128, 1)
    rowid = i * 128 + lax.broadcasted_iota(jnp.int32, (128, 1), 0)
    npts = sfull_ref.shape[1]
    acc = jnp.zeros((128, 1), jnp.float32)
    for c in range(npts // 1280):
        sc = sfull_ref[0, c * 1280:(c + 1) * 1280].reshape(1, 1280)
        colid = c * 1280 + lax.broadcasted_iota(jnp.int32, (128, 1280), 1)
        gt = (sc > s_r).astype(jnp.float32)
        eq = jnp.logical_and(sc == s_r, colid < rowid).astype(jnp.float32)
        acc = acc + jnp.sum(gt + eq, axis=1, keepdims=True)
    o_ref[...] = acc.reshape(1, 1, 128)


def _rank_all(s, npad):
    s_pad = jnp.concatenate([s, jnp.full((npad - s.shape[0],), -1.0, jnp.float32)])
    nr = npad // 128
    out = pl.pallas_call(
        _rank_kernel,
        grid=(nr,),
        in_specs=[pl.BlockSpec((1, npad), lambda i: (0, 0)),
                  pl.BlockSpec((1, 1, 128), lambda i: (i, 0, 0))],
        out_specs=pl.BlockSpec((1, 1, 128), lambda i: (i, 0, 0)),
        out_shape=jax.ShapeDtypeStruct((nr, 1, 128), jnp.float32),
    )(s_pad.reshape(1, npad), s_pad.reshape(nr, 1, 128))
    return out.reshape(npad)[:s.shape[0]].astype(jnp.int32)


def _onehot_rows_kernel(m_ref, o_ref):
    blk_r, width = o_ref.shape
    m = m_ref[0, 0, :]
    cols = jax.lax.broadcasted_iota(jnp.int32, (blk_r, width), 1)
    o_ref[...] = (cols == m[:, None]).astype(jnp.float32)


def _emit_onehot(m, n_rows, n_cols, blk_r):
    grid = n_rows // blk_r
    return pl.pallas_call(
        _onehot_rows_kernel,
        grid=(grid,),
        in_specs=[pl.BlockSpec((1, 1, blk_r), lambda i: (i, 0, 0))],
        out_specs=pl.BlockSpec((blk_r, n_cols), lambda i: (i, 0)),
        out_shape=jax.ShapeDtypeStruct((n_rows, n_cols), jnp.float32),
    )(m.reshape(grid, 1, blk_r))


def kernel(x, edge_index, batch, epoch_id, W0, b0, W1, b1, p0, p1, gate, Wl, bl):
    src = jnp.concatenate([edge_index[0], jnp.array([N - 1], dtype=edge_index.dtype)])
    dst = jnp.concatenate([edge_index[1], jnp.array([N - 1], dtype=edge_index.dtype)])
    w = jnp.ones((src.shape[0],), jnp.float32)

    npad = EP - src.shape[0]
    padbins = (N + (jnp.arange(npad, dtype=jnp.int32) % 240)).astype(jnp.int32)
    src_pad = jnp.concatenate([src, padbins])
    dst_pad = jnp.concatenate([dst, padbins])

    degp = _deg_partials(dst_pad)
    deg0 = jnp.clip(degp[0, :N] + degp[1, :N], 1.0, None)
    norm0 = w / jnp.sqrt(deg0[src] * deg0[dst])

    h0 = x @ W0
    agg0 = jax.ops.segment_sum(h0[src] * norm0[:, None], dst, num_segments=N)
    e1 = jax.nn.relu(agg0 + b0)

    k0, k1 = 5000, 2500
    score0 = jax.nn.sigmoid(e1 @ p0)
    rank0 = _rank_all(score0, DEGB)
    perm0, vals0 = _perm_and_vals(rank0, score0, k0)
    x1 = e1[perm0] * vals0[:, None]
    x1 = _normalize(x1, axis=1)
    x1 = x1 / jnp.clip(x1.sum(1, keepdims=True), 1.0, None)

    keptm = (rank0 < k0).astype(jnp.float32)
    m0z = jnp.where(rank0 < k0, rank0, 0).astype(jnp.int32)
    keptm_ext = jnp.concatenate([keptm, jnp.zeros((DEGB - N,), jnp.float32)])
    m0z_ext = jnp.concatenate(
        [m0z, jnp.full((DEGB - N,), k0 + 1, jnp.int32)])
    w1p, src1p, dst1p, deg1p = _relabel_edges(src_pad, dst_pad, keptm_ext,
                                              m0z_ext)
    ne = src.shape[0]
    w1 = w1p.reshape(EP)[:ne]
    src1 = src1p.reshape(EP)[:ne]
    dst1 = dst1p.reshape(EP)[:ne]
    deg1 = jnp.clip(deg1p[0, :k0] + deg1p[1, :k0], 1.0, None)
    norm1 = w1 / jnp.sqrt(deg1[src1] * deg1[dst1])

    h1 = x1 @ W1
    agg1 = jax.ops.segment_sum(h1[src1] * norm1[:, None], dst1,
                               num_segments=k0)
    e2 = jax.nn.relu(agg1 + b1)

    score1 = jax.nn.sigmoid(e2 @ p1)
    rank1 = _rank_all(score1, 5120)

    m0 = jnp.where(rank0 < k0, rank0, -1).astype(jnp.int32)
    rec0 = _emit_onehot(m0, N, k0, 200)
    m1 = jnp.where(rank1 < k1, rank1, -1).astype(jnp.int32)
    rec1 = _emit_onehot(m1, k0, k1, 200)

    e2z = jnp.concatenate([e2, jnp.zeros((8, HID), jnp.float32)], axis=0)
    nid = jnp.arange(DEGB, dtype=jnp.int32)
    m0g_ext = jnp.where(
        jnp.concatenate([rank0, jnp.full((DEGB - N,), 2 * DEGB, jnp.int32)])
        < k0,
        jnp.concatenate([rank0, jnp.zeros((DEGB - N,), jnp.int32)]),
        k0 + (nid & 7))
    e2_up = _gather_rows(e2z, m0g_ext)[:N]

    alpha = jax.nn.softmax(gate)
    invs = 1.0 / jnp.sqrt(deg0)
    h2p, lp = _merge_scale(e1, e2_up, Wl, invs, alpha)
    loss_kl = -0.5 * (jnp.sum(lp) / N)
    h2p_ext = jnp.concatenate(
        [h2p, jnp.zeros((DEGB - N, HID), jnp.float32)], axis=0)
    aggp = _row_scatter_sum(h2p_ext, src_pad, dst_pad)
    out = _finalize(aggp[0, :N], aggp[1, :N], invs, bl)
    return out, loss_kl, rec0, rec1, alpha


# speedup vs baseline: 1.2318x; 1.2318x over previous
import functools

import jax
import jax.numpy as jnp
import numpy as np
from jax import lax
from jax.experimental import pallas as pl
from jax.experimental.pallas import tpu as pltpu
from jax.experimental.pallas import tpu_sc as plsc

N = 10000
E = 160000
FEAT = 128
HID = 128
RATIO = 0.5

NW = 32
EP = 163840
EPW = EP // NW
DEGB = 10240

def _sc_mesh():
    return plsc.VectorSubcoreMesh(core_axis_name="c", subcore_axis_name="s")


def _sc_wid():
    return lax.axis_index("c") * 16 + lax.axis_index("s")


def _deg_sc_kernel(dst_hbm, out_hbm, dstv, onesv, zbuf, degsh):
    c = lax.axis_index("c")
    s = lax.axis_index("s")
    wid = c * 16 + s

    def zfill(i, _):
        zbuf[pl.ds(i * 16, 16)] = jnp.zeros((16,), jnp.float32)
        return 0
    lax.fori_loop(0, zbuf.shape[0] // 16, zfill, 0)

    def ofill(i, _):
        onesv[pl.ds(i * 16, 16)] = jnp.ones((16,), jnp.float32)
        return 0
    lax.fori_loop(0, EPW // 16, ofill, 0)

    sl = DEGB // 16
    pltpu.sync_copy(zbuf, degsh.at[pl.ds(s * sl, sl)])
    pltpu.sync_copy(dst_hbm.at[wid], dstv)
    plsc.subcore_barrier()
    pltpu.sync_copy(onesv, degsh.at[dstv], add=True)
    plsc.subcore_barrier()
    pltpu.sync_copy(degsh.at[pl.ds(s * sl, sl)], out_hbm.at[c, pl.ds(s * sl, sl)])


def _deg_partials(dst_pad):
    k = pl.kernel(
        _deg_sc_kernel,
        out_type=jax.ShapeDtypeStruct((2, DEGB), jnp.float32),
        mesh=_sc_mesh(),
        scratch_types=[
            pltpu.VMEM((EPW,), jnp.int32),
            pltpu.VMEM((EPW,), jnp.float32),
            pltpu.VMEM((DEGB // 16,), jnp.float32),
            pltpu.VMEM_SHARED((DEGB,), jnp.float32),
        ],
    )
    return k(dst_pad.reshape(NW, EPW))


def _eprod_sc_kernel(src_hbm, dst_hbm, tab_hbm, out_hbm, sv, dv, av, bv, ov,
                     tabsh):
    wid = _sc_wid()
    s = lax.axis_index("s")
    sl = DEGB // 16
    pltpu.sync_copy(tab_hbm.at[pl.ds(s * sl, sl)], tabsh.at[pl.ds(s * sl, sl)])
    pltpu.sync_copy(src_hbm.at[wid], sv)
    pltpu.sync_copy(dst_hbm.at[wid], dv)
    plsc.subcore_barrier()
    pltpu.sync_copy(tabsh.at[sv], av)
    pltpu.sync_copy(tabsh.at[dv], bv)

    def body(i, _):
        sl = pl.ds(i * 16, 16)
        ov[sl] = av[sl] * bv[sl]
        return 0
    lax.fori_loop(0, EPW // 16, body, 0)
    pltpu.sync_copy(ov, out_hbm.at[wid])


def _edge_products(src_pad, dst_pad, table):
    k = pl.kernel(
        _eprod_sc_kernel,
        out_type=jax.ShapeDtypeStruct((NW, EPW), jnp.float32),
        mesh=_sc_mesh(),
        scratch_types=[
            pltpu.VMEM((EPW,), jnp.int32),
            pltpu.VMEM((EPW,), jnp.int32),
            pltpu.VMEM((EPW,), jnp.float32),
            pltpu.VMEM((EPW,), jnp.float32),
            pltpu.VMEM((EPW,), jnp.float32),
            pltpu.VMEM_SHARED((DEGB,), jnp.float32),
        ],
    )
    return k(src_pad.reshape(NW, EPW), dst_pad.reshape(NW, EPW), table)


def _permvals_sc_kernel(rank_hbm, s_hbm, nid_hbm, perm_hbm, vals_hbm,
                        rv, sv, nv, iv):
    wid = _sc_wid()
    kslots = perm_hbm.shape[0] - 120
    pltpu.sync_copy(rank_hbm.at[wid], rv)
    pltpu.sync_copy(s_hbm.at[wid], sv)
    pltpu.sync_copy(nid_hbm.at[wid], nv)

    def body(i, _):
        sl = pl.ds(i * 16, 16)
        r = rv[sl]
        nid = nv[sl]
        idx = jnp.where(r < kslots, r, kslots + lax.rem(nid, 120))
        rv[sl] = idx
        return 0
    lax.fori_loop(0, rv.shape[0] // 16, body, 0)
    pltpu.sync_copy(nv, perm_hbm.at[rv])
    pltpu.sync_copy(sv, vals_hbm.at[rv])


def _perm_and_vals(rank, s, k):
    npd = rank.shape[0] if rank.shape[0] % NW == 0 else DEGB
    chunk = DEGB // NW
    rank_p = jnp.concatenate(
        [rank, jnp.full((DEGB - rank.shape[0],), 2 * DEGB, jnp.int32)])
    s_p = jnp.concatenate(
        [s, jnp.zeros((DEGB - s.shape[0],), jnp.float32)])
    nid = jnp.arange(DEGB, dtype=jnp.int32)
    kp = pl.kernel(
        _permvals_sc_kernel,
        out_type=(jax.ShapeDtypeStruct((k + 120,), jnp.int32),
                  jax.ShapeDtypeStruct((k + 120,), jnp.float32)),
        mesh=_sc_mesh(),
        scratch_types=[
            pltpu.VMEM((chunk,), jnp.int32),
            pltpu.VMEM((chunk,), jnp.float32),
            pltpu.VMEM((chunk,), jnp.int32),
            pltpu.VMEM((chunk,), jnp.int32),
        ],
    )
    perm_p, vals_p = kp(rank_p.reshape(NW, chunk), s_p.reshape(NW, chunk),
                        nid.reshape(NW, chunk))
    return perm_p[:k], vals_p[:k]


def _relabel_sc_kernel(src_hbm, dst_hbm, keptm_hbm, m0z_hbm,
                       w1_hbm, src1_hbm, dst1_hbm, degout_hbm,
                       sv, dv, ks, kd, ms, md, wv, zbuf, degsh, keptsh, mzsh):
    c = lax.axis_index("c")
    s = lax.axis_index("s")
    wid = c * 16 + s

    def zfill(i, _):
        zbuf[pl.ds(i * 16, 16)] = jnp.zeros((16,), jnp.float32)
        return 0
    lax.fori_loop(0, zbuf.shape[0] // 16, zfill, 0)
    sl = DEGB // 16
    pltpu.sync_copy(zbuf, degsh.at[pl.ds(s * sl, sl)])
    pltpu.sync_copy(keptm_hbm.at[pl.ds(s * sl, sl)],
                    keptsh.at[pl.ds(s * sl, sl)])
    pltpu.sync_copy(m0z_hbm.at[pl.ds(s * sl, sl)], mzsh.at[pl.ds(s * sl, sl)])

    pltpu.sync_copy(src_hbm.at[wid], sv)
    pltpu.sync_copy(dst_hbm.at[wid], dv)
    plsc.subcore_barrier()
    pltpu.sync_copy(keptsh.at[sv], ks)
    pltpu.sync_copy(keptsh.at[dv], kd)
    pltpu.sync_copy(mzsh.at[sv], ms)
    pltpu.sync_copy(mzsh.at[dv], md)

    def body(i, _):
        q = pl.ds(i * 16, 16)
        wv[q] = ks[q] * kd[q]
        return 0
    lax.fori_loop(0, EPW // 16, body, 0)
    plsc.subcore_barrier()
    pltpu.sync_copy(wv, degsh.at[md], add=True)
    pltpu.sync_copy(wv, w1_hbm.at[wid])
    pltpu.sync_copy(ms, src1_hbm.at[wid])
    pltpu.sync_copy(md, dst1_hbm.at[wid])
    plsc.subcore_barrier()
    pltpu.sync_copy(degsh.at[pl.ds(s * sl, sl)], degout_hbm.at[c, pl.ds(s * sl, sl)])


def _relabel_edges(src_pad, dst_pad, keptm_ext, m0z_ext):
    k = pl.kernel(
        _relabel_sc_kernel,
        out_type=(jax.ShapeDtypeStruct((NW, EPW), jnp.float32),
                  jax.ShapeDtypeStruct((NW, EPW), jnp.int32),
                  jax.ShapeDtypeStruct((NW, EPW), jnp.int32),
                  jax.ShapeDtypeStruct((2, DEGB), jnp.float32)),
        mesh=_sc_mesh(),
        scratch_types=[
            pltpu.VMEM((EPW,), jnp.int32),
            pltpu.VMEM((EPW,), jnp.int32),
            pltpu.VMEM((EPW,), jnp.float32),
            pltpu.VMEM((EPW,), jnp.float32),
            pltpu.VMEM((EPW,), jnp.int32),
            pltpu.VMEM((EPW,), jnp.int32),
            pltpu.VMEM((EPW,), jnp.float32),
            pltpu.VMEM((DEGB // 16,), jnp.float32),
            pltpu.VMEM_SHARED((DEGB,), jnp.float32),
            pltpu.VMEM_SHARED((DEGB,), jnp.float32),
            pltpu.VMEM_SHARED((DEGB,), jnp.int32),
        ],
    )
    return k(src_pad.reshape(NW, EPW), dst_pad.reshape(NW, EPW),
             keptm_ext, m0z_ext)


def _gather_rows_sc_kernel(tab_hbm, idx_hbm, out_hbm, iv, rows, sem):
    wid = _sc_wid()
    chunk = iv.shape[0]
    pltpu.sync_copy(idx_hbm.at[wid], iv)
    pltpu.async_copy(tab_hbm.at[iv], rows, sem).wait()
    pltpu.sync_copy(rows, out_hbm.at[pl.ds(wid * chunk, chunk)])


def _gather_rows(tab, idx_ext):
    chunk = idx_ext.shape[0] // NW
    k = pl.kernel(
        _gather_rows_sc_kernel,
        out_type=jax.ShapeDtypeStruct((idx_ext.shape[0], HID), jnp.float32),
        mesh=_sc_mesh(),
        scratch_types=[
            pltpu.VMEM((chunk,), jnp.int32),
            pltpu.VMEM((chunk, HID), jnp.float32),
            pltpu.SemaphoreType.DMA,
        ],
    )
    return k(tab, idx_ext.reshape(NW, chunk))


def _rowagg_sc_kernel(h_hbm, src_hbm, dst_hbm, out_hbm,
                      srcv, dstv, rows, zbuf, accsh):
    c = lax.axis_index("c")
    s = lax.axis_index("s")
    wid = c * 16 + s

    def zfill(i, _):
        zbuf[i // 8, pl.ds((i % 8) * 16, 16)] = jnp.zeros((16,), jnp.float32)
        return 0
    lax.fori_loop(0, 16 * 8, zfill, 0)

    def zcopy(i, _):
        pltpu.sync_copy(zbuf, accsh.at[pl.ds(s * 640 + i * 16, 16)])
        return 0
    lax.fori_loop(0, 40, zcopy, 0)

    pltpu.sync_copy(src_hbm.at[wid], srcv)
    pltpu.sync_copy(dst_hbm.at[wid], dstv)
    plsc.subcore_barrier()

    def chunk(i, _):
        pltpu.sync_copy(h_hbm.at[srcv.at[i]], rows)
        pltpu.sync_copy(rows, accsh.at[dstv.at[i]], add=True)
        return 0
    lax.fori_loop(0, srcv.shape[0], chunk, 0)
    plsc.subcore_barrier()
    pltpu.sync_copy(accsh.at[pl.ds(s * 640, 640)],
                    out_hbm.at[c, pl.ds(s * 640, 640)])


def _row_scatter_sum(h_ext, src_pad, dst_pad):
    nchunk, csz = 40, 128
    k = pl.kernel(
        _rowagg_sc_kernel,
        out_type=jax.ShapeDtypeStruct((2, DEGB, HID), jnp.float32),
        mesh=_sc_mesh(),
        scratch_types=[
            pltpu.VMEM((nchunk, csz), jnp.int32),
            pltpu.VMEM((nchunk, csz), jnp.int32),
            pltpu.VMEM((csz, HID), jnp.float32),
            pltpu.VMEM((16, HID), jnp.float32),
            pltpu.VMEM_SHARED((DEGB, HID), jnp.float32),
        ],
    )
    return k(h_ext, src_pad.reshape(NW, nchunk, csz),
             dst_pad.reshape(NW, nchunk, csz))


def _bigrows_sc_kernel(tab_hbm, idx_hbm, out_hbm, idxv, rows):
    wid = _sc_wid()
    pltpu.sync_copy(idx_hbm.at[wid], idxv)

    def chunk(j, _):
        pltpu.sync_copy(tab_hbm.at[idxv.at[j]], rows)
        pltpu.sync_copy(rows, out_hbm.at[pl.ds(wid * EPW + j * 128, 128)])
        return 0
    lax.fori_loop(0, idxv.shape[0], chunk, 0)


def _gather_rows_big(tab_ext, idx_pad):
    k = pl.kernel(
        _bigrows_sc_kernel,
        out_type=jax.ShapeDtypeStruct((EP, HID), jnp.float32),
        mesh=_sc_mesh(),
        scratch_types=[
            pltpu.VMEM((EPW // 128, 128), jnp.int32),
            pltpu.VMEM((128, HID), jnp.float32),
        ],
    )
    return k(tab_ext, idx_pad.reshape(NW, EPW // 128, 128))


def _merge_kernel(e1_ref, e2u_ref, wl_ref, invs_ref, alpha_ref,
                  h2p_ref, lp_ref):
    a0 = alpha_ref[0]
    a1 = alpha_ref[1]
    e1b = e1_ref[...]
    emb = a0 * e1b + a1 * e2u_ref[...]
    lp_ref[...] = jnp.sum(1.0 + e1b - emb * emb - jnp.exp(e1b), axis=0,
                          keepdims=True).reshape(1, 1, HID)
    h2 = jnp.dot(emb, wl_ref[...], preferred_element_type=jnp.float32)
    h2p_ref[...] = h2 * invs_ref[0, 0, :].reshape(e1b.shape[0], 1)


def _merge_scale(e1, e2_up, Wl, invs, alpha, blk=400):
    grid = N // blk
    h2p, lp = pl.pallas_call(
        _merge_kernel,
        grid=(grid,),
        in_specs=[pl.BlockSpec((blk, HID), lambda i: (i, 0)),
                  pl.BlockSpec((blk, HID), lambda i: (i, 0)),
                  pl.BlockSpec((HID, HID), lambda i: (0, 0)),
                  pl.BlockSpec((1, 1, blk), lambda i: (i, 0, 0)),
                  pl.BlockSpec(memory_space=pltpu.SMEM)],
        out_specs=[pl.BlockSpec((blk, HID), lambda i: (i, 0)),
                   pl.BlockSpec((1, 1, HID), lambda i: (i, 0, 0))],
        out_shape=[jax.ShapeDtypeStruct((N, HID), jnp.float32),
                   jax.ShapeDtypeStruct((grid, 1, HID), jnp.float32)],
    )(e1, e2_up, Wl, invs.reshape(grid, 1, blk), alpha)
    return h2p, lp


def _finalize_kernel(pa_ref, pb_ref, invs_ref, bl_ref, o_ref):
    blk = o_ref.shape[0]
    v = ((pa_ref[...] + pb_ref[...]) * invs_ref[0, 0, :].reshape(blk, 1)
         + bl_ref[...])
    n = jnp.sqrt(jnp.sum(v * v, axis=1, keepdims=True))
    o_ref[...] = v / jnp.maximum(n, 1e-12)


def _finalize(pa, pb, invs, bl, blk=400):
    grid = N // blk
    return pl.pallas_call(
        _finalize_kernel,
        grid=(grid,),
        in_specs=[pl.BlockSpec((blk, HID), lambda i: (i, 0)),
                  pl.BlockSpec((blk, HID), lambda i: (i, 0)),
                  pl.BlockSpec((1, 1, blk), lambda i: (i, 0, 0)),
                  pl.BlockSpec((1, HID), lambda i: (0, 0))],
        out_specs=pl.BlockSpec((blk, HID), lambda i: (i, 0)),
        out_shape=jax.ShapeDtypeStruct((N, HID), jnp.float32),
    )(pa, pb, invs.reshape(grid, 1, blk), bl.reshape(1, HID))


def _normalize(x, axis=-1, eps=1e-12):
    n = jnp.linalg.norm(x, axis=axis, keepdims=True)
    return x / jnp.maximum(n, eps)


def _gcn(x, W, b, src, dst, w, n):
    h = x @ W
    deg = jax.ops.segment_sum(w, dst, num_segments=n)
    deg = jnp.clip(deg, 1.0, None)
    norm = w / jnp.sqrt(deg[src] * deg[dst])
    agg = jax.ops.segment_sum(h[src] * norm[:, None], dst, num_segments=n)
    return agg + b


def _pool_meta(emb, p, n, ratio):
    score = jax.nn.sigmoid(emb @ p)
    k = int(np.ceil(ratio * n))
    vals, perm = jax.lax.top_k(score, k)
    kept = jnp.zeros((n,), jnp.float32).at[perm].set(1.0)
    mapping = jnp.zeros((n,), jnp.int32).at[perm].set(jnp.arange(k, dtype=jnp.int32))
    return vals, perm, kept, mapping, k


def _rank_kernel(sfull_ref, srow_ref, o_ref):
    i = pl.program_id(0)
    s_r = srow_ref[0, 0, :].reshape(128, 1)
    rowid = i * 128 + lax.broadcasted_iota(jnp.int32, (128, 1), 0)
    npts = sfull_ref.shape[1]
    acc = jnp.zeros((128, 1), jnp.float32)
    for c in range(npts // 1280):
        sc = sfull_ref[0, c * 1280:(c + 1) * 1280].reshape(1, 1280)
        colid = c * 1280 + lax.broadcasted_iota(jnp.int32, (128, 1280), 1)
        gt = (sc > s_r).astype(jnp.float32)
        eq = jnp.logical_and(sc == s_r, colid < rowid).astype(jnp.float32)
        acc = acc + jnp.sum(gt + eq, axis=1, keepdims=True)
    o_ref[...] = acc.reshape(1, 1, 128)


def _rank_all(s, npad):
    s_pad = jnp.concatenate([s, jnp.full((npad - s.shape[0],), -1.0, jnp.float32)])
    nr = npad // 128
    out = pl.pallas_call(
        _rank_kernel,
        grid=(nr,),
        in_specs=[pl.BlockSpec((1, npad), lambda i: (0, 0)),
                  pl.BlockSpec((1, 1, 128), lambda i: (i, 0, 0))],
        out_specs=pl.BlockSpec((1, 1, 128), lambda i: (i, 0, 0)),
        out_shape=jax.ShapeDtypeStruct((nr, 1, 128), jnp.float32),
    )(s_pad.reshape(1, npad), s_pad.reshape(nr, 1, 128))
    return out.reshape(npad)[:s.shape[0]].astype(jnp.int32)


def _onehot_rows_kernel(m_ref, o_ref):
    blk_r, width = o_ref.shape
    m = m_ref[0, 0, :]
    cols = jax.lax.broadcasted_iota(jnp.int32, (blk_r, width), 1)
    o_ref[...] = (cols == m[:, None]).astype(jnp.float32)


def _emit_onehot(m, n_rows, n_cols, blk_r):
    grid = n_rows // blk_r
    return pl.pallas_call(
        _onehot_rows_kernel,
        grid=(grid,),
        in_specs=[pl.BlockSpec((1, 1, blk_r), lambda i: (i, 0, 0))],
        out_specs=pl.BlockSpec((blk_r, n_cols), lambda i: (i, 0)),
        out_shape=jax.ShapeDtypeStruct((n_rows, n_cols), jnp.float32),
    )(m.reshape(grid, 1, blk_r))


def kernel(x, edge_index, batch, epoch_id, W0, b0, W1, b1, p0, p1, gate, Wl, bl):
    src = jnp.concatenate([edge_index[0], jnp.array([N - 1], dtype=edge_index.dtype)])
    dst = jnp.concatenate([edge_index[1], jnp.array([N - 1], dtype=edge_index.dtype)])
    w = jnp.ones((src.shape[0],), jnp.float32)

    npad = EP - src.shape[0]
    padbins = (N + (jnp.arange(npad, dtype=jnp.int32) % 240)).astype(jnp.int32)
    src_pad = jnp.concatenate([src, padbins])
    dst_pad = jnp.concatenate([dst, padbins])

    degp = _deg_partials(dst_pad)
    deg0 = jnp.clip(degp[0, :N] + degp[1, :N], 1.0, None)
    deg0_ext = jnp.concatenate([deg0, jnp.ones((DEGB - N,), jnp.float32)])
    prod0 = _edge_products(src_pad, dst_pad, deg0_ext).reshape(EP)[:src.shape[0]]
    norm0 = w / jnp.sqrt(prod0)

    h0 = x @ W0
    h0_ext = jnp.concatenate([h0, jnp.zeros((DEGB - N, HID), jnp.float32)])
    hs0 = _gather_rows_big(h0_ext, src_pad)[:src.shape[0]]
    agg0 = jax.ops.segment_sum(hs0 * norm0[:, None], dst, num_segments=N)
    e1 = jax.nn.relu(agg0 + b0)

    k0, k1 = 5000, 2500
    score0 = jax.nn.sigmoid(e1 @ p0)
    rank0 = _rank_all(score0, DEGB)
    perm0, vals0 = _perm_and_vals(rank0, score0, k0)
    x1 = e1[perm0] * vals0[:, None]
    x1 = _normalize(x1, axis=1)
    x1 = x1 / jnp.clip(x1.sum(1, keepdims=True), 1.0, None)

    keptm = (rank0 < k0).astype(jnp.float32)
    m0z = jnp.where(rank0 < k0, rank0, 0).astype(jnp.int32)
    keptm_ext = jnp.concatenate([keptm, jnp.zeros((DEGB - N,), jnp.float32)])
    m0z_ext = jnp.concatenate(
        [m0z, jnp.full((DEGB - N,), k0 + 1, jnp.int32)])
    w1p, src1p, dst1p, deg1p = _relabel_edges(src_pad, dst_pad, keptm_ext,
                                              m0z_ext)
    ne = src.shape[0]
    w1 = w1p.reshape(EP)[:ne]
    src1 = src1p.reshape(EP)[:ne]
    dst1 = dst1p.reshape(EP)[:ne]
    deg1 = jnp.clip(deg1p[0, :k0] + deg1p[1, :k0], 1.0, None)
    deg1_ext = jnp.concatenate([deg1, jnp.ones((DEGB - k0,), jnp.float32)])
    prod1 = _edge_products(src1p.reshape(EP), dst1p.reshape(EP),
                           deg1_ext).reshape(EP)[:ne]
    norm1 = w1 / jnp.sqrt(prod1)

    h1 = x1 @ W1
    h1_ext = jnp.concatenate([h1, jnp.zeros((DEGB - k0, HID), jnp.float32)])
    hs1 = _gather_rows_big(h1_ext, src1p.reshape(EP))[:ne]
    agg1 = jax.ops.segment_sum(hs1 * norm1[:, None], dst1, num_segments=k0)
    e2 = jax.nn.relu(agg1 + b1)

    score1 = jax.nn.sigmoid(e2 @ p1)
    rank1 = _rank_all(score1, 5120)

    m0 = jnp.where(rank0 < k0, rank0, -1).astype(jnp.int32)
    rec0 = _emit_onehot(m0, N, k0, 200)
    m1 = jnp.where(rank1 < k1, rank1, -1).astype(jnp.int32)
    rec1 = _emit_onehot(m1, k0, k1, 200)

    e2z = jnp.concatenate([e2, jnp.zeros((8, HID), jnp.float32)], axis=0)
    nid = jnp.arange(DEGB, dtype=jnp.int32)
    m0g_ext = jnp.where(
        jnp.concatenate([rank0, jnp.full((DEGB - N,), 2 * DEGB, jnp.int32)])
        < k0,
        jnp.concatenate([rank0, jnp.zeros((DEGB - N,), jnp.int32)]),
        k0 + (nid & 7))
    e2_up = _gather_rows(e2z, m0g_ext)[:N]

    alpha = jax.nn.softmax(gate)
    invs = 1.0 / jnp.sqrt(deg0)
    h2p, lp = _merge_scale(e1, e2_up, Wl, invs, alpha)
    loss_kl = -0.5 * (jnp.sum(lp) / N)
    h2p_ext = jnp.concatenate(
        [h2p, jnp.zeros((DEGB - N, HID), jnp.float32)], axis=0)
    aggp = _row_scatter_sum(h2p_ext, src_pad, dst_pad)
    out = _finalize(aggp[0, :N], aggp[1, :N], invs, bl)
    return out, loss_kl, rec0, rec1, alpha


# speedup vs baseline: 1.9633x; 1.5938x over previous
import functools

import jax
import jax.numpy as jnp
import numpy as np
from jax import lax
from jax.experimental import pallas as pl
from jax.experimental.pallas import tpu as pltpu
from jax.experimental.pallas import tpu_sc as plsc

N = 10000
E = 160000
FEAT = 128
HID = 128
RATIO = 0.5

NW = 32
EP = 163840
EPW = EP // NW
DEGB = 10240

def _sc_mesh():
    return plsc.VectorSubcoreMesh(core_axis_name="c", subcore_axis_name="s")


def _sc_wid():
    return lax.axis_index("c") * 16 + lax.axis_index("s")


def _deg_sc_kernel(dst_hbm, out_hbm, dstv, onesv, zbuf, degsh):
    c = lax.axis_index("c")
    s = lax.axis_index("s")
    wid = c * 16 + s

    def zfill(i, _):
        zbuf[pl.ds(i * 16, 16)] = jnp.zeros((16,), jnp.float32)
        return 0
    lax.fori_loop(0, zbuf.shape[0] // 16, zfill, 0)

    def ofill(i, _):
        onesv[pl.ds(i * 16, 16)] = jnp.ones((16,), jnp.float32)
        return 0
    lax.fori_loop(0, EPW // 16, ofill, 0)

    sl = DEGB // 16
    pltpu.sync_copy(zbuf, degsh.at[pl.ds(s * sl, sl)])
    pltpu.sync_copy(dst_hbm.at[wid], dstv)
    plsc.subcore_barrier()
    pltpu.sync_copy(onesv, degsh.at[dstv], add=True)
    plsc.subcore_barrier()
    pltpu.sync_copy(degsh.at[pl.ds(s * sl, sl)], out_hbm.at[c, pl.ds(s * sl, sl)])


def _deg_partials(dst_pad):
    k = pl.kernel(
        _deg_sc_kernel,
        out_type=jax.ShapeDtypeStruct((2, DEGB), jnp.float32),
        mesh=_sc_mesh(),
        scratch_types=[
            pltpu.VMEM((EPW,), jnp.int32),
            pltpu.VMEM((EPW,), jnp.float32),
            pltpu.VMEM((DEGB // 16,), jnp.float32),
            pltpu.VMEM_SHARED((DEGB,), jnp.float32),
        ],
    )
    return k(dst_pad.reshape(NW, EPW))


def _eprod_sc_kernel(src_hbm, dst_hbm, tab_hbm, out_hbm, sv, dv, av, bv, ov,
                     tabsh):
    wid = _sc_wid()
    s = lax.axis_index("s")
    sl = DEGB // 16
    pltpu.sync_copy(tab_hbm.at[pl.ds(s * sl, sl)], tabsh.at[pl.ds(s * sl, sl)])
    pltpu.sync_copy(src_hbm.at[wid], sv)
    pltpu.sync_copy(dst_hbm.at[wid], dv)
    plsc.subcore_barrier()
    pltpu.sync_copy(tabsh.at[sv], av)
    pltpu.sync_copy(tabsh.at[dv], bv)

    def body(i, _):
        sl = pl.ds(i * 16, 16)
        ov[sl] = av[sl] * bv[sl]
        return 0
    lax.fori_loop(0, EPW // 16, body, 0)
    pltpu.sync_copy(ov, out_hbm.at[wid])


def _edge_products(src_pad, dst_pad, table):
    k = pl.kernel(
        _eprod_sc_kernel,
        out_type=jax.ShapeDtypeStruct((NW, EPW), jnp.float32),
        mesh=_sc_mesh(),
        scratch_types=[
            pltpu.VMEM((EPW,), jnp.int32),
            pltpu.VMEM((EPW,), jnp.int32),
            pltpu.VMEM((EPW,), jnp.float32),
            pltpu.VMEM((EPW,), jnp.float32),
            pltpu.VMEM((EPW,), jnp.float32),
            pltpu.VMEM_SHARED((DEGB,), jnp.float32),
        ],
    )
    return k(src_pad.reshape(NW, EPW), dst_pad.reshape(NW, EPW), table)


def _permvals_sc_kernel(rank_hbm, s_hbm, nid_hbm, perm_hbm, vals_hbm,
                        rv, sv, nv, iv):
    wid = _sc_wid()
    kslots = perm_hbm.shape[0] - 120
    pltpu.sync_copy(rank_hbm.at[wid], rv)
    pltpu.sync_copy(s_hbm.at[wid], sv)
    pltpu.sync_copy(nid_hbm.at[wid], nv)

    def body(i, _):
        sl = pl.ds(i * 16, 16)
        r = rv[sl]
        nid = nv[sl]
        idx = jnp.where(r < kslots, r, kslots + lax.rem(nid, 120))
        rv[sl] = idx
        return 0
    lax.fori_loop(0, rv.shape[0] // 16, body, 0)
    pltpu.sync_copy(nv, perm_hbm.at[rv])
    pltpu.sync_copy(sv, vals_hbm.at[rv])


def _perm_and_vals(rank, s, k):
    chunk = DEGB // NW
    rank_p = jnp.concatenate(
        [rank, jnp.full((DEGB - rank.shape[0],), 2 * DEGB, jnp.int32)])
    s_p = jnp.concatenate(
        [s, jnp.zeros((DEGB - s.shape[0],), jnp.float32)])
    nid = jnp.arange(DEGB, dtype=jnp.int32)
    kp = pl.kernel(
        _permvals_sc_kernel,
        out_type=(jax.ShapeDtypeStruct((k + 120,), jnp.int32),
                  jax.ShapeDtypeStruct((k + 120,), jnp.float32)),
        mesh=_sc_mesh(),
        scratch_types=[
            pltpu.VMEM((chunk,), jnp.int32),
            pltpu.VMEM((chunk,), jnp.float32),
            pltpu.VMEM((chunk,), jnp.int32),
            pltpu.VMEM((chunk,), jnp.int32),
        ],
    )
    perm_p, vals_p = kp(rank_p.reshape(NW, chunk), s_p.reshape(NW, chunk),
                        nid.reshape(NW, chunk))
    return perm_p[:k], vals_p[:k]


def _relabel_sc_kernel(src_hbm, dst_hbm, keptm_hbm, m0z_hbm,
                       w1_hbm, src1_hbm, dst1_hbm, degout_hbm,
                       sv, dv, ks, kd, ms, md, wv, zbuf, degsh, keptsh, mzsh):
    c = lax.axis_index("c")
    s = lax.axis_index("s")
    wid = c * 16 + s

    def zfill(i, _):
        zbuf[pl.ds(i * 16, 16)] = jnp.zeros((16,), jnp.float32)
        return 0
    lax.fori_loop(0, zbuf.shape[0] // 16, zfill, 0)
    sl = DEGB // 16
    pltpu.sync_copy(zbuf, degsh.at[pl.ds(s * sl, sl)])
    pltpu.sync_copy(keptm_hbm.at[pl.ds(s * sl, sl)],
                    keptsh.at[pl.ds(s * sl, sl)])
    pltpu.sync_copy(m0z_hbm.at[pl.ds(s * sl, sl)], mzsh.at[pl.ds(s * sl, sl)])

    pltpu.sync_copy(src_hbm.at[wid], sv)
    pltpu.sync_copy(dst_hbm.at[wid], dv)
    plsc.subcore_barrier()
    pltpu.sync_copy(keptsh.at[sv], ks)
    pltpu.sync_copy(keptsh.at[dv], kd)
    pltpu.sync_copy(mzsh.at[sv], ms)
    pltpu.sync_copy(mzsh.at[dv], md)

    def body(i, _):
        q = pl.ds(i * 16, 16)
        wv[q] = ks[q] * kd[q]
        return 0
    lax.fori_loop(0, EPW // 16, body, 0)
    plsc.subcore_barrier()
    pltpu.sync_copy(wv, degsh.at[md], add=True)
    pltpu.sync_copy(wv, w1_hbm.at[wid])
    pltpu.sync_copy(ms, src1_hbm.at[wid])
    pltpu.sync_copy(md, dst1_hbm.at[wid])
    plsc.subcore_barrier()
    pltpu.sync_copy(degsh.at[pl.ds(s * sl, sl)], degout_hbm.at[c, pl.ds(s * sl, sl)])


def _relabel_edges(src_pad, dst_pad, keptm_ext, m0z_ext):
    k = pl.kernel(
        _relabel_sc_kernel,
        out_type=(jax.ShapeDtypeStruct((NW, EPW), jnp.float32),
                  jax.ShapeDtypeStruct((NW, EPW), jnp.int32),
                  jax.ShapeDtypeStruct((NW, EPW), jnp.int32),
                  jax.ShapeDtypeStruct((2, DEGB), jnp.float32)),
        mesh=_sc_mesh(),
        scratch_types=[
            pltpu.VMEM((EPW,), jnp.int32),
            pltpu.VMEM((EPW,), jnp.int32),
            pltpu.VMEM((EPW,), jnp.float32),
            pltpu.VMEM((EPW,), jnp.float32),
            pltpu.VMEM((EPW,), jnp.int32),
            pltpu.VMEM((EPW,), jnp.int32),
            pltpu.VMEM((EPW,), jnp.float32),
            pltpu.VMEM((DEGB // 16,), jnp.float32),
            pltpu.VMEM_SHARED((DEGB,), jnp.float32),
            pltpu.VMEM_SHARED((DEGB,), jnp.float32),
            pltpu.VMEM_SHARED((DEGB,), jnp.int32),
        ],
    )
    return k(src_pad.reshape(NW, EPW), dst_pad.reshape(NW, EPW),
             keptm_ext, m0z_ext)


def _gather_rows_sc_kernel(tab_hbm, idx_hbm, out_hbm, iv, rows, sem):
    wid = _sc_wid()
    chunk = iv.shape[0]
    pltpu.sync_copy(idx_hbm.at[wid], iv)
    pltpu.async_copy(tab_hbm.at[iv], rows, sem).wait()
    pltpu.sync_copy(rows, out_hbm.at[pl.ds(wid * chunk, chunk)])


def _gather_rows(tab, idx_ext):
    chunk = idx_ext.shape[0] // NW
    k = pl.kernel(
        _gather_rows_sc_kernel,
        out_type=jax.ShapeDtypeStruct((idx_ext.shape[0], HID), jnp.float32),
        mesh=_sc_mesh(),
        scratch_types=[
            pltpu.VMEM((chunk,), jnp.int32),
            pltpu.VMEM((chunk, HID), jnp.float32),
            pltpu.SemaphoreType.DMA,
        ],
    )
    return k(tab, idx_ext.reshape(NW, chunk))


def _rowagg_sc_kernel(h_hbm, src_hbm, dst_hbm, out_hbm,
                      srcv, dstv, rows, zbuf, accsh):
    c = lax.axis_index("c")
    s = lax.axis_index("s")
    wid = c * 16 + s

    def zfill(i, _):
        zbuf[i // 8, pl.ds((i % 8) * 16, 16)] = jnp.zeros((16,), jnp.float32)
        return 0
    lax.fori_loop(0, 16 * 8, zfill, 0)

    def zcopy(i, _):
        pltpu.sync_copy(zbuf, accsh.at[pl.ds(s * 640 + i * 16, 16)])
        return 0
    lax.fori_loop(0, 40, zcopy, 0)

    pltpu.sync_copy(src_hbm.at[wid], srcv)
    pltpu.sync_copy(dst_hbm.at[wid], dstv)
    plsc.subcore_barrier()

    def chunk(i, _):
        pltpu.sync_copy(h_hbm.at[srcv.at[i]], rows)
        pltpu.sync_copy(rows, accsh.at[dstv.at[i]], add=True)
        return 0
    lax.fori_loop(0, srcv.shape[0], chunk, 0)
    plsc.subcore_barrier()
    pltpu.sync_copy(accsh.at[pl.ds(s * 640, 640)],
                    out_hbm.at[c, pl.ds(s * 640, 640)])


def _row_scatter_sum(h_ext, src_pad, dst_pad):
    nchunk, csz = 40, 128
    k = pl.kernel(
        _rowagg_sc_kernel,
        out_type=jax.ShapeDtypeStruct((2, DEGB, HID), jnp.float32),
        mesh=_sc_mesh(),
        scratch_types=[
            pltpu.VMEM((nchunk, csz), jnp.int32),
            pltpu.VMEM((nchunk, csz), jnp.int32),
            pltpu.VMEM((csz, HID), jnp.float32),
            pltpu.VMEM((16, HID), jnp.float32),
            pltpu.VMEM_SHARED((DEGB, HID), jnp.float32),
        ],
    )
    return k(h_ext, src_pad.reshape(NW, nchunk, csz),
             dst_pad.reshape(NW, nchunk, csz))


def _merge_kernel(e1_ref, e2u_ref, wl_ref, invs_ref, alpha_ref,
                  h2p_ref, lp_ref):
    a0 = alpha_ref[0]
    a1 = alpha_ref[1]
    e1b = e1_ref[...]
    emb = a0 * e1b + a1 * e2u_ref[...]
    lp_ref[...] = jnp.sum(1.0 + e1b - emb * emb - jnp.exp(e1b), axis=0,
                          keepdims=True).reshape(1, 1, HID)
    h2 = jnp.dot(emb, wl_ref[...], preferred_element_type=jnp.float32)
    h2p_ref[...] = h2 * invs_ref[0, 0, :].reshape(e1b.shape[0], 1)


def _merge_scale(e1, e2_up, Wl, invs, alpha, blk=400):
    grid = N // blk
    h2p, lp = pl.pallas_call(
        _merge_kernel,
        grid=(grid,),
        in_specs=[pl.BlockSpec((blk, HID), lambda i: (i, 0)),
                  pl.BlockSpec((blk, HID), lambda i: (i, 0)),
                  pl.BlockSpec((HID, HID), lambda i: (0, 0)),
                  pl.BlockSpec((1, 1, blk), lambda i: (i, 0, 0)),
                  pl.BlockSpec(memory_space=pltpu.SMEM)],
        out_specs=[pl.BlockSpec((blk, HID), lambda i: (i, 0)),
                   pl.BlockSpec((1, 1, HID), lambda i: (i, 0, 0))],
        out_shape=[jax.ShapeDtypeStruct((N, HID), jnp.float32),
                   jax.ShapeDtypeStruct((grid, 1, HID), jnp.float32)],
    )(e1, e2_up, Wl, invs.reshape(grid, 1, blk), alpha)
    return h2p, lp


def _finalize_kernel(pa_ref, pb_ref, invs_ref, bl_ref, o_ref):
    blk = o_ref.shape[0]
    v = ((pa_ref[...] + pb_ref[...]) * invs_ref[0, 0, :].reshape(blk, 1)
         + bl_ref[...])
    n = jnp.sqrt(jnp.sum(v * v, axis=1, keepdims=True))
    o_ref[...] = v / jnp.maximum(n, 1e-12)


def _finalize(pa, pb, invs, bl, blk=400):
    grid = N // blk
    return pl.pallas_call(
        _finalize_kernel,
        grid=(grid,),
        in_specs=[pl.BlockSpec((blk, HID), lambda i: (i, 0)),
                  pl.BlockSpec((blk, HID), lambda i: (i, 0)),
                  pl.BlockSpec((1, 1, blk), lambda i: (i, 0, 0)),
                  pl.BlockSpec((1, HID), lambda i: (0, 0))],
        out_specs=pl.BlockSpec((blk, HID), lambda i: (i, 0)),
        out_shape=jax.ShapeDtypeStruct((N, HID), jnp.float32),
    )(pa, pb, invs.reshape(grid, 1, blk), bl.reshape(1, HID))


def _normalize(x, axis=-1, eps=1e-12):
    n = jnp.linalg.norm(x, axis=axis, keepdims=True)
    return x / jnp.maximum(n, eps)


def _rank_kernel(sfull_ref, srow_ref, o_ref):
    i = pl.program_id(0)
    s_r = srow_ref[0, 0, :].reshape(128, 1)
    rowid = i * 128 + lax.broadcasted_iota(jnp.int32, (128, 1), 0)
    npts = sfull_ref.shape[1]
    acc = jnp.zeros((128, 1), jnp.float32)
    for c in range(npts // 1280):
        sc = sfull_ref[0, c * 1280:(c + 1) * 1280].reshape(1, 1280)
        colid = c * 1280 + lax.broadcasted_iota(jnp.int32, (128, 1280), 1)
        gt = (sc > s_r).astype(jnp.float32)
        eq = jnp.logical_and(sc == s_r, colid < rowid).astype(jnp.float32)
        acc = acc + jnp.sum(gt + eq, axis=1, keepdims=True)
    o_ref[...] = acc.reshape(1, 1, 128)


def _rank_all(s, npad):
    s_pad = jnp.concatenate([s, jnp.full((npad - s.shape[0],), -1.0, jnp.float32)])
    nr = npad // 128
    out = pl.pallas_call(
        _rank_kernel,
        grid=(nr,),
        in_specs=[pl.BlockSpec((1, npad), lambda i: (0, 0)),
                  pl.BlockSpec((1, 1, 128), lambda i: (i, 0, 0))],
        out_specs=pl.BlockSpec((1, 1, 128), lambda i: (i, 0, 0)),
        out_shape=jax.ShapeDtypeStruct((nr, 1, 128), jnp.float32),
    )(s_pad.reshape(1, npad), s_pad.reshape(nr, 1, 128))
    return out.reshape(npad)[:s.shape[0]].astype(jnp.int32)


def _onehot_rows_kernel(m_ref, o_ref):
    blk_r, width = o_ref.shape
    m = m_ref[0, 0, :]
    cols = jax.lax.broadcasted_iota(jnp.int32, (blk_r, width), 1)
    o_ref[...] = (cols == m[:, None]).astype(jnp.float32)


def _emit_onehot(m, n_rows, n_cols, blk_r):
    grid = n_rows // blk_r
    return pl.pallas_call(
        _onehot_rows_kernel,
        grid=(grid,),
        in_specs=[pl.BlockSpec((1, 1, blk_r), lambda i: (i, 0, 0))],
        out_specs=pl.BlockSpec((blk_r, n_cols), lambda i: (i, 0)),
        out_shape=jax.ShapeDtypeStruct((n_rows, n_cols), jnp.float32),
    )(m.reshape(grid, 1, blk_r))


def kernel(x, edge_index, batch, epoch_id, W0, b0, W1, b1, p0, p1, gate, Wl, bl):
    src = jnp.concatenate([edge_index[0], jnp.array([N - 1], dtype=edge_index.dtype)])
    dst = jnp.concatenate([edge_index[1], jnp.array([N - 1], dtype=edge_index.dtype)])
    w = jnp.ones((src.shape[0],), jnp.float32)

    npad = EP - src.shape[0]
    padbins = (N + (jnp.arange(npad, dtype=jnp.int32) % 240)).astype(jnp.int32)
    src_pad = jnp.concatenate([src, padbins])
    dst_pad = jnp.concatenate([dst, padbins])

    degp = _deg_partials(dst_pad)
    deg0 = jnp.clip(degp[0, :N] + degp[1, :N], 1.0, None)
    deg0_ext = jnp.concatenate([deg0, jnp.ones((DEGB - N,), jnp.float32)])
    prod0 = _edge_products(src_pad, dst_pad, deg0_ext).reshape(EP)[:src.shape[0]]
    norm0 = w / jnp.sqrt(prod0)

    h0 = x @ W0
    agg0 = jax.ops.segment_sum(h0[src] * norm0[:, None], dst, num_segments=N)
    e1 = jax.nn.relu(agg0 + b0)

    k0, k1 = 5000, 2500
    score0 = jax.nn.sigmoid(e1 @ p0)
    rank0 = _rank_all(score0, DEGB)
    perm0, vals0 = _perm_and_vals(rank0, score0, k0)
    x1 = e1[perm0] * vals0[:, None]
    x1 = _normalize(x1, axis=1)
    x1 = x1 / jnp.clip(x1.sum(1, keepdims=True), 1.0, None)

    keptm = (rank0 < k0).astype(jnp.float32)
    m0z = jnp.where(rank0 < k0, rank0, 0).astype(jnp.int32)
    keptm_ext = jnp.concatenate([keptm, jnp.zeros((DEGB - N,), jnp.float32)])
    m0z_ext = jnp.concatenate(
        [m0z, jnp.full((DEGB - N,), k0 + 1, jnp.int32)])
    w1p, src1p, dst1p, deg1p = _relabel_edges(src_pad, dst_pad, keptm_ext,
                                              m0z_ext)
    ne = src.shape[0]
    w1 = w1p.reshape(EP)[:ne]
    src1 = src1p.reshape(EP)[:ne]
    dst1 = dst1p.reshape(EP)[:ne]
    deg1 = jnp.clip(deg1p[0, :k0] + deg1p[1, :k0], 1.0, None)
    deg1_ext = jnp.concatenate([deg1, jnp.ones((DEGB - k0,), jnp.float32)])
    prod1 = _edge_products(src1p.reshape(EP), dst1p.reshape(EP),
                           deg1_ext).reshape(EP)[:ne]
    norm1 = w1 / jnp.sqrt(prod1)

    h1 = x1 @ W1
    agg1 = jax.ops.segment_sum(h1[src1] * norm1[:, None], dst1,
                               num_segments=k0)
    e2 = jax.nn.relu(agg1 + b1)

    score1 = jax.nn.sigmoid(e2 @ p1)
    rank1 = _rank_all(score1, 5120)

    m0 = jnp.where(rank0 < k0, rank0, -1).astype(jnp.int32)
    rec0 = _emit_onehot(m0, N, k0, 200)
    m1 = jnp.where(rank1 < k1, rank1, -1).astype(jnp.int32)
    rec1 = _emit_onehot(m1, k0, k1, 200)

    e2z = jnp.concatenate([e2, jnp.zeros((8, HID), jnp.float32)], axis=0)
    nid = jnp.arange(DEGB, dtype=jnp.int32)
    m0g_ext = jnp.where(
        jnp.concatenate([rank0, jnp.full((DEGB - N,), 2 * DEGB, jnp.int32)])
        < k0,
        jnp.concatenate([rank0, jnp.zeros((DEGB - N,), jnp.int32)]),
        k0 + (nid & 7))
    e2_up = _gather_rows(e2z, m0g_ext)[:N]

    alpha = jax.nn.softmax(gate)
    invs = 1.0 / jnp.sqrt(deg0)
    h2p, lp = _merge_scale(e1, e2_up, Wl, invs, alpha)
    loss_kl = -0.5 * (jnp.sum(lp) / N)
    h2p_ext = jnp.concatenate(
        [h2p, jnp.zeros((DEGB - N, HID), jnp.float32)], axis=0)
    aggp = _row_scatter_sum(h2p_ext, src_pad, dst_pad)
    out = _finalize(aggp[0, :N], aggp[1, :N], invs, bl)
    return out, loss_kl, rec0, rec1, alpha
